# baseline (speedup 1.0000x reference)
import dataclasses

import jax
import jax.numpy as jnp
from jax.experimental import pallas as pl
from jax.experimental.pallas import tpu as pltpu
from jax.experimental.pallas import tpu_sc as plsc

_B = 128
_N = 8192
_K = 64
_OUT = 16
_TEMP = 0.5
_V = 16
_NCHUNK = _N // _V
_SUBS_PER_CORE = _OUT // 2


def _sc_gate_kernel(scores_ref, out_ref, col_ref, candb_ref,
                    candi_ref, gate_ref, sem):
    c = jax.lax.axis_index("c")
    s = jax.lax.axis_index("s")

    @pl.when(s < _SUBS_PER_CORE)
    def _active():
        j = c * _SUBS_PER_CORE + s
        pltpu.async_copy(scores_ref.at[j], col_ref, sem).wait()

        lanes = jax.lax.iota(jnp.int32, _V)

        def _maxpass(i, carry):
            a0, a1, a2, a3 = carry
            base = i * 4 * _V
            b0 = plsc.bitcast(col_ref[pl.ds(base, _V)], jnp.int32)
            b1 = plsc.bitcast(col_ref[pl.ds(base + _V, _V)], jnp.int32)
            b2 = plsc.bitcast(col_ref[pl.ds(base + 2 * _V, _V)],
                              jnp.int32)
            b3 = plsc.bitcast(col_ref[pl.ds(base + 3 * _V, _V)],
                              jnp.int32)
            return (jnp.maximum(a0, b0), jnp.maximum(a1, b1),
                    jnp.maximum(a2, b2), jnp.maximum(a3, b3))

        z = jnp.zeros((_V,), jnp.int32)
        a0, a1, a2, a3 = jax.lax.fori_loop(0, _NCHUNK // 4, _maxpass,
                                           (z, z, z, z))
        t_c = jnp.min(jnp.minimum(jnp.minimum(a0, a1),
                                  jnp.minimum(a2, a3)))
        gmax = jnp.max(jnp.maximum(jnp.maximum(a0, a1),
                                   jnp.maximum(a2, a3)))

        def _count_ge(ref, n_lim, t):
            def body(i, acc):
                v = ref[pl.ds(i * _V, _V)]
                valid = lanes + i * _V < n_lim
                return acc + jnp.sum(
                    ((v >= t) & valid).astype(jnp.int32))
            nch = (n_lim + _V - 1) // _V
            return jax.lax.fori_loop(0, nch, body, jnp.int32(0))

        def _kth_largest(ref, n_lim, kth, lo, hi):
            def cond(carry):
                lo, hi = carry
                return hi - lo > 1

            def body(carry):
                lo, hi = carry
                mid = (lo + hi) >> 1
                ge = _count_ge(ref, n_lim, mid) >= kth
                return (jnp.where(ge, mid, lo), jnp.where(ge, hi, mid))

            lo, hi = jax.lax.while_loop(cond, body, (lo, hi))
            return lo

        def _compact(i, off):
            v = col_ref[pl.ds(i * _V, _V)]
            bits = plsc.bitcast(v, jnp.int32)
            m = bits >= t_c
            idxv = lanes + i * _V
            plsc.store_compressed(candb_ref.at[pl.ds(off, _V)], bits,
                                  mask=m)
            plsc.store_compressed(candi_ref.at[pl.ds(off, _V)], idxv,
                                  mask=m)
            return off + jnp.sum(m.astype(jnp.int32))

        n_cand = jax.lax.fori_loop(0, _NCHUNK, _compact, jnp.int32(0),
                                   unroll=4)

        tau = _kth_largest(candb_ref, n_cand, jnp.int32(_K), t_c,
                           gmax + 1)

        def _count_gt(i, acc):
            v = candb_ref[pl.ds(i * _V, _V)]
            valid = lanes + i * _V < n_cand
            return acc + jnp.sum(((v > tau) & valid).astype(jnp.int32))

        ncch = (n_cand + _V - 1) // _V
        n_gt = jax.lax.fori_loop(0, ncch, _count_gt, jnp.int32(0))
        need = _K - n_gt

        def cond2(carry):
            lo, hi = carry
            return hi - lo > 1

        def body2(carry):
            lo, hi = carry
            mid = (lo + hi) >> 1

            def cnt_body(i, acc):
                v = candb_ref[pl.ds(i * _V, _V)]
                ix = candi_ref[pl.ds(i * _V, _V)]
                valid = lanes + i * _V < n_cand
                m = (v == tau) & (ix < mid) & valid
                return acc + jnp.sum(m.astype(jnp.int32))

            cnt = jax.lax.fori_loop(0, ncch, cnt_body, jnp.int32(0))
            ok = cnt <= need
            return (jnp.where(ok, mid, lo), jnp.where(ok, hi, mid))

        cut, _ = jax.lax.while_loop(cond2, body2,
                                    (jnp.int32(0), jnp.int32(_N + 1)))

        def _gatepass(i, acc):
            v = col_ref[pl.ds(i * _V, _V)]
            bits = plsc.bitcast(v, jnp.int32)
            idxv = lanes + i * _V
            kept = (bits > tau) | ((bits == tau) & (idxv < cut))
            e = jnp.where(kept, jnp.exp(v * (1.0 / _TEMP)), 0.0)
            gate_ref[pl.ds(i * _V, _V)] = e
            return acc + e

        acc = jax.lax.fori_loop(0, _NCHUNK, _gatepass,
                                jnp.zeros((_V,), jnp.float32), unroll=4)
        denom = jnp.broadcast_to(jnp.sum(acc), (_V,))
        inv = jnp.ones((_V,), jnp.float32) / denom

        def _normpass(i, _):
            base = i * _V
            gate_ref[pl.ds(base, _V)] = gate_ref[pl.ds(base, _V)] * inv
            return 0

        jax.lax.fori_loop(0, _NCHUNK, _normpass, 0, unroll=4)
        pltpu.async_copy(gate_ref, out_ref.at[j], sem).wait()


def _sc_gate(scores_t):
    cp = pltpu.CompilerParams()
    if "needs_layout_passes" in pltpu.CompilerParams.__dataclass_fields__:
        cp = dataclasses.replace(cp, needs_layout_passes=False)
    kfn = pl.kernel(
        _sc_gate_kernel,
        out_type=jax.ShapeDtypeStruct((_OUT, _N), jnp.float32),
        mesh=plsc.VectorSubcoreMesh(core_axis_name="c",
                                    subcore_axis_name="s"),
        compiler_params=cp,
        scratch_types=[
            pltpu.VMEM((_N,), jnp.float32),
            pltpu.VMEM((_N + _V,), jnp.int32),
            pltpu.VMEM((_N + _V,), jnp.int32),
            pltpu.VMEM((_N,), jnp.float32),
            pltpu.SemaphoreType.DMA,
        ],
    )
    return kfn(scores_t)


def _bcast_kernel(gate_ref, out_ref):
    out_ref[...] = gate_ref[...][None]


def kernel(X, scores):
    del X
    gate_t = _sc_gate(scores.T)
    out_t = pl.pallas_call(
        _bcast_kernel,
        grid=(_B,),
        in_specs=[pl.BlockSpec((_OUT, _N), lambda b: (0, 0))],
        out_specs=pl.BlockSpec((1, _OUT, _N), lambda b: (b, 0, 0)),
        out_shape=jax.ShapeDtypeStruct((_B, _OUT, _N), jnp.float32),
        compiler_params=pltpu.CompilerParams(
            dimension_semantics=("parallel",)),
    )(gate_t)
    return out_t.transpose(0, 2, 1)


# speedup vs baseline: 1.6985x; 1.6985x over previous
import dataclasses

import jax
import jax.numpy as jnp
from jax.experimental import pallas as pl
from jax.experimental.pallas import tpu as pltpu
from jax.experimental.pallas import tpu_sc as plsc

_B = 128
_N = 8192
_K = 64
_OUT = 16
_TEMP = 0.5
_V = 16
_NCHUNK = _N // _V
_SUBS_PER_CORE = _OUT // 2


def _sc_gate_kernel(scores_ref, out_ref, col_ref, candb_ref,
                    candi_ref, gate_ref, sem):
    c = jax.lax.axis_index("c")
    s = jax.lax.axis_index("s")

    @pl.when(s < _SUBS_PER_CORE)
    def _active():
        j = c * _SUBS_PER_CORE + s
        pltpu.async_copy(scores_ref.at[j], col_ref, sem).wait()

        lanes = jax.lax.iota(jnp.int32, _V)

        def _maxpass(i, carry):
            a0, a1, a2, a3 = carry
            base = i * 4 * _V
            b0 = plsc.bitcast(col_ref[pl.ds(base, _V)], jnp.int32)
            b1 = plsc.bitcast(col_ref[pl.ds(base + _V, _V)], jnp.int32)
            b2 = plsc.bitcast(col_ref[pl.ds(base + 2 * _V, _V)],
                              jnp.int32)
            b3 = plsc.bitcast(col_ref[pl.ds(base + 3 * _V, _V)],
                              jnp.int32)
            return (jnp.maximum(a0, b0), jnp.maximum(a1, b1),
                    jnp.maximum(a2, b2), jnp.maximum(a3, b3))

        z = jnp.zeros((_V,), jnp.int32)
        a0, a1, a2, a3 = jax.lax.fori_loop(0, _NCHUNK // 4, _maxpass,
                                           (z, z, z, z))
        t_c = jnp.min(jnp.minimum(jnp.minimum(a0, a1),
                                  jnp.minimum(a2, a3)))
        gmax = jnp.max(jnp.maximum(jnp.maximum(a0, a1),
                                   jnp.maximum(a2, a3)))

        def _count_ge(ref, n_lim, t):
            def body(i, acc):
                v = ref[pl.ds(i * _V, _V)]
                valid = lanes + i * _V < n_lim
                return acc + ((v >= t) & valid).astype(jnp.int32)

            nch = (n_lim + _V - 1) // _V
            accv = jax.lax.fori_loop(0, nch, body,
                                     jnp.zeros((_V,), jnp.int32))
            return jnp.sum(accv)

        def _kth_largest(ref, n_lim, kth, lo, hi):
            def cond(carry):
                lo, hi = carry
                return hi - lo > 1

            def body(carry):
                lo, hi = carry
                mid = (lo + hi) >> 1
                ge = _count_ge(ref, n_lim, mid) >= kth
                return (jnp.where(ge, mid, lo), jnp.where(ge, hi, mid))

            lo, hi = jax.lax.while_loop(cond, body, (lo, hi))
            return lo

        def _compact(i, off):
            v = col_ref[pl.ds(i * _V, _V)]
            bits = plsc.bitcast(v, jnp.int32)
            m = bits >= t_c
            idxv = lanes + i * _V
            plsc.store_compressed(candb_ref.at[pl.ds(off, _V)], bits,
                                  mask=m)
            plsc.store_compressed(candi_ref.at[pl.ds(off, _V)], idxv,
                                  mask=m)
            return off + jnp.sum(m.astype(jnp.int32))

        n_cand = jax.lax.fori_loop(0, _NCHUNK, _compact, jnp.int32(0),
                                   unroll=4)

        tau = _kth_largest(candb_ref, n_cand, jnp.int32(_K), t_c,
                           gmax + 1)

        ncch = (n_cand + _V - 1) // _V

        def _count_gt(i, acc):
            v = candb_ref[pl.ds(i * _V, _V)]
            valid = lanes + i * _V < n_cand
            return acc + ((v > tau) & valid).astype(jnp.int32)

        n_gt = jnp.sum(jax.lax.fori_loop(0, ncch, _count_gt,
                                         jnp.zeros((_V,), jnp.int32)))
        need = _K - n_gt

        def cond2(carry):
            lo, hi = carry
            return hi - lo > 1

        def body2(carry):
            lo, hi = carry
            mid = (lo + hi) >> 1

            def cnt_body(i, acc):
                v = candb_ref[pl.ds(i * _V, _V)]
                ix = candi_ref[pl.ds(i * _V, _V)]
                valid = lanes + i * _V < n_cand
                m = (v == tau) & (ix < mid) & valid
                return acc + m.astype(jnp.int32)

            cnt = jnp.sum(jax.lax.fori_loop(0, ncch, cnt_body,
                                            jnp.zeros((_V,), jnp.int32)))
            ok = cnt <= need
            return (jnp.where(ok, mid, lo), jnp.where(ok, hi, mid))

        cut, _ = jax.lax.while_loop(cond2, body2,
                                    (jnp.int32(0), jnp.int32(_N + 1)))

        def _zero(i, _):
            gate_ref[pl.ds(i * _V, _V)] = jnp.zeros((_V,), jnp.float32)
            return 0

        jax.lax.fori_loop(0, _NCHUNK, _zero, 0, unroll=4)

        def _kept(i):
            v = candb_ref[pl.ds(i * _V, _V)]
            ix = candi_ref[pl.ds(i * _V, _V)]
            valid = lanes + i * _V < n_cand
            kept = ((v > tau) | ((v == tau) & (ix < cut))) & valid
            e = jnp.exp(plsc.bitcast(v, jnp.float32) * (1.0 / _TEMP))
            return ix, kept, e

        def _denom_body(i, acc):
            _, kept, e = _kept(i)
            return acc + jnp.where(kept, e, 0.0)

        acc = jax.lax.fori_loop(0, ncch, _denom_body,
                                jnp.zeros((_V,), jnp.float32))
        inv = jnp.ones((_V,), jnp.float32) / jnp.broadcast_to(
            jnp.sum(acc), (_V,))

        def _scatter_body(i, _):
            ix, kept, e = _kept(i)
            plsc.store_scatter(gate_ref, [ix], e * inv, mask=kept)
            return 0

        jax.lax.fori_loop(0, ncch, _scatter_body, 0)
        pltpu.async_copy(gate_ref, out_ref.at[j], sem).wait()


def _sc_gate(scores_t):
    cp = pltpu.CompilerParams()
    if "needs_layout_passes" in pltpu.CompilerParams.__dataclass_fields__:
        cp = dataclasses.replace(cp, needs_layout_passes=False)
    kfn = pl.kernel(
        _sc_gate_kernel,
        out_type=jax.ShapeDtypeStruct((_OUT, _N), jnp.float32),
        mesh=plsc.VectorSubcoreMesh(core_axis_name="c",
                                    subcore_axis_name="s"),
        compiler_params=cp,
        scratch_types=[
            pltpu.VMEM((_N,), jnp.float32),
            pltpu.VMEM((_N + _V,), jnp.int32),
            pltpu.VMEM((_N + _V,), jnp.int32),
            pltpu.VMEM((_N,), jnp.float32),
            pltpu.SemaphoreType.DMA,
        ],
    )
    return kfn(scores_t)


def _bcast_kernel(gate_ref, out_ref, sem):
    def _start(b, _):
        pltpu.make_async_copy(gate_ref, out_ref.at[b], sem).start()
        return 0

    jax.lax.fori_loop(0, _B, _start, 0)

    def _wait(b, _):
        pltpu.make_async_copy(gate_ref, out_ref.at[b], sem).wait()
        return 0

    jax.lax.fori_loop(0, _B, _wait, 0)


def kernel(X, scores):
    del X
    gate_t = _sc_gate(scores.T)
    out_t = pl.pallas_call(
        _bcast_kernel,
        in_specs=[pl.BlockSpec(memory_space=pltpu.VMEM)],
        out_specs=pl.BlockSpec(memory_space=pl.ANY),
        out_shape=jax.ShapeDtypeStruct((_B, _OUT, _N), jnp.float32),
        scratch_shapes=[pltpu.SemaphoreType.DMA],
    )(gate_t)
    return out_t.transpose(0, 2, 1)


# speedup vs baseline: 1.7104x; 1.0070x over previous
import dataclasses

import jax
import jax.numpy as jnp
from jax.experimental import pallas as pl
from jax.experimental.pallas import tpu as pltpu
from jax.experimental.pallas import tpu_sc as plsc

_B = 128
_N = 8192
_K = 64
_OUT = 16
_TEMP = 0.5
_V = 16
_NCHUNK = _N // _V
_SUBS_PER_CORE = _OUT // 2


def _sc_gate_kernel(scores_ref, out_ref, col_ref, candb_ref,
                    candi_ref, gate_ref, sem):
    c = jax.lax.axis_index("c")
    s = jax.lax.axis_index("s")

    @pl.when(c == 0)
    def _active():
        j = s
        pltpu.async_copy(scores_ref.at[j], col_ref, sem).wait()

        lanes = jax.lax.iota(jnp.int32, _V)

        def _maxpass(i, carry):
            a0, a1, a2, a3 = carry
            base = i * 4 * _V
            b0 = plsc.bitcast(col_ref[pl.ds(base, _V)], jnp.int32)
            b1 = plsc.bitcast(col_ref[pl.ds(base + _V, _V)], jnp.int32)
            b2 = plsc.bitcast(col_ref[pl.ds(base + 2 * _V, _V)],
                              jnp.int32)
            b3 = plsc.bitcast(col_ref[pl.ds(base + 3 * _V, _V)],
                              jnp.int32)
            return (jnp.maximum(a0, b0), jnp.maximum(a1, b1),
                    jnp.maximum(a2, b2), jnp.maximum(a3, b3))

        z = jnp.zeros((_V,), jnp.int32)
        a0, a1, a2, a3 = jax.lax.fori_loop(0, _NCHUNK // 4, _maxpass,
                                           (z, z, z, z))
        t_c = jnp.min(jnp.minimum(jnp.minimum(a0, a1),
                                  jnp.minimum(a2, a3)))
        gmax = jnp.max(jnp.maximum(jnp.maximum(a0, a1),
                                   jnp.maximum(a2, a3)))

        def _count_ge(ref, n_lim, t):
            def body(i, acc):
                v = ref[pl.ds(i * _V, _V)]
                valid = lanes + i * _V < n_lim
                return acc + ((v >= t) & valid).astype(jnp.int32)

            nch = (n_lim + _V - 1) // _V
            accv = jax.lax.fori_loop(0, nch, body,
                                     jnp.zeros((_V,), jnp.int32))
            return jnp.sum(accv)

        def _kth_largest(ref, n_lim, kth, lo, hi):
            def cond(carry):
                lo, hi = carry
                return hi - lo > 1

            def body(carry):
                lo, hi = carry
                mid = (lo + hi) >> 1
                ge = _count_ge(ref, n_lim, mid) >= kth
                return (jnp.where(ge, mid, lo), jnp.where(ge, hi, mid))

            lo, hi = jax.lax.while_loop(cond, body, (lo, hi))
            return lo

        def _compact(i, off):
            v = col_ref[pl.ds(i * _V, _V)]
            bits = plsc.bitcast(v, jnp.int32)
            m = bits >= t_c
            idxv = lanes + i * _V
            plsc.store_compressed(candb_ref.at[pl.ds(off, _V)], bits,
                                  mask=m)
            plsc.store_compressed(candi_ref.at[pl.ds(off, _V)], idxv,
                                  mask=m)
            return off + jnp.sum(m.astype(jnp.int32))

        n_cand = jax.lax.fori_loop(0, _NCHUNK, _compact, jnp.int32(0),
                                   unroll=4)

        tau = _kth_largest(candb_ref, n_cand, jnp.int32(_K), t_c,
                           gmax + 1)

        ncch = (n_cand + _V - 1) // _V

        def _count_gt(i, acc):
            v = candb_ref[pl.ds(i * _V, _V)]
            valid = lanes + i * _V < n_cand
            return acc + ((v > tau) & valid).astype(jnp.int32)

        n_gt = jnp.sum(jax.lax.fori_loop(0, ncch, _count_gt,
                                         jnp.zeros((_V,), jnp.int32)))
        need = _K - n_gt

        def cond2(carry):
            lo, hi = carry
            return hi - lo > 1

        def body2(carry):
            lo, hi = carry
            mid = (lo + hi) >> 1

            def cnt_body(i, acc):
                v = candb_ref[pl.ds(i * _V, _V)]
                ix = candi_ref[pl.ds(i * _V, _V)]
                valid = lanes + i * _V < n_cand
                m = (v == tau) & (ix < mid) & valid
                return acc + m.astype(jnp.int32)

            cnt = jnp.sum(jax.lax.fori_loop(0, ncch, cnt_body,
                                            jnp.zeros((_V,), jnp.int32)))
            ok = cnt <= need
            return (jnp.where(ok, mid, lo), jnp.where(ok, hi, mid))

        cut, _ = jax.lax.while_loop(cond2, body2,
                                    (jnp.int32(0), jnp.int32(_N + 1)))

        def _zero(i, _):
            gate_ref[pl.ds(i * _V, _V)] = jnp.zeros((_V,), jnp.float32)
            return 0

        jax.lax.fori_loop(0, _NCHUNK, _zero, 0, unroll=4)

        def _kept(i):
            v = candb_ref[pl.ds(i * _V, _V)]
            ix = candi_ref[pl.ds(i * _V, _V)]
            valid = lanes + i * _V < n_cand
            kept = ((v > tau) | ((v == tau) & (ix < cut))) & valid
            e = jnp.exp(plsc.bitcast(v, jnp.float32) * (1.0 / _TEMP))
            return ix, kept, e

        def _denom_body(i, acc):
            _, kept, e = _kept(i)
            return acc + jnp.where(kept, e, 0.0)

        acc = jax.lax.fori_loop(0, ncch, _denom_body,
                                jnp.zeros((_V,), jnp.float32))
        inv = jnp.ones((_V,), jnp.float32) / jnp.broadcast_to(
            jnp.sum(acc), (_V,))

        def _scatter_body(i, _):
            ix, kept, e = _kept(i)
            plsc.store_scatter(gate_ref, [ix], e * inv, mask=kept)
            return 0

        jax.lax.fori_loop(0, ncch, _scatter_body, 0)
        pltpu.async_copy(gate_ref, out_ref.at[j], sem).wait()


def _sc_gate(scores_t):
    cp = pltpu.CompilerParams()
    if "needs_layout_passes" in pltpu.CompilerParams.__dataclass_fields__:
        cp = dataclasses.replace(cp, needs_layout_passes=False)
    kfn = pl.kernel(
        _sc_gate_kernel,
        out_type=jax.ShapeDtypeStruct((_OUT, _N), jnp.float32),
        mesh=plsc.VectorSubcoreMesh(core_axis_name="c",
                                    subcore_axis_name="s"),
        compiler_params=cp,
        scratch_types=[
            pltpu.VMEM((_N,), jnp.float32),
            pltpu.VMEM((_N + _V,), jnp.int32),
            pltpu.VMEM((_N + _V,), jnp.int32),
            pltpu.VMEM((_N,), jnp.float32),
            pltpu.SemaphoreType.DMA,
        ],
    )
    return kfn(scores_t)


def _bcast_kernel(gate_ref, out_ref, sem):
    def _start(b, _):
        pltpu.make_async_copy(gate_ref, out_ref.at[b], sem).start()
        return 0

    jax.lax.fori_loop(0, _B, _start, 0)

    def _wait(b, _):
        pltpu.make_async_copy(gate_ref, out_ref.at[b], sem).wait()
        return 0

    jax.lax.fori_loop(0, _B, _wait, 0)


def kernel(X, scores):
    del X
    gate_t = _sc_gate(scores.T)
    out_t = pl.pallas_call(
        _bcast_kernel,
        in_specs=[pl.BlockSpec(memory_space=pltpu.VMEM)],
        out_specs=pl.BlockSpec(memory_space=pl.ANY),
        out_shape=jax.ShapeDtypeStruct((_B, _OUT, _N), jnp.float32),
        scratch_shapes=[pltpu.SemaphoreType.DMA],
    )(gate_t)
    return out_t.transpose(0, 2, 1)
